# Initial kernel scaffold; baseline (speedup 1.0000x reference)
import jax
import jax.numpy as jnp
from jax.experimental import pallas as pl


def kernel(x_feat, clustering_labels, edge_index, batch, W0, b0, W1, b1, W2, b2, M1, mb1, M2, mb2):
    raise NotImplementedError("write your pallas kernel here")


# speedup vs baseline: 59.7041x; 59.7041x over previous
import functools

import jax
import jax.numpy as jnp
from jax import lax
from jax.experimental import pallas as pl
from jax.experimental.pallas import tpu as pltpu
from jax.experimental.pallas import tpu_sc as plsc

N = 10000
E = 320000
C = 8
IN = 128
H = 256
OUT = 128
G = 16

NPAD = 10240
EPAD = 327680
BN = 512
NB = NPAD // BN
CH = 128
NTILE = 16
RPT = NPAD // NTILE

_SC_MESH = dict(core_axis_name="c", subcore_axis_name="s")


def _zero_buf(buf, d):
    def zrow(r, _):
        for j in range(d // 16):
            buf[r, pl.ds(j * 16, 16)] = jnp.zeros((16,), jnp.float32)
        return 0

    lax.fori_loop(0, CH, zrow, 0)


def _zero_acc(buf, acc_sh, sid):
    base_r = sid * RPT
    for j in range(RPT // CH):
        pltpu.sync_copy(buf, acc_sh.at[pl.ds(base_r + j * CH, CH)])


def _make_deg_kernel():
    ept = EPAD // 2 // NTILE
    nchunk = ept // CH

    @functools.partial(
        pl.kernel,
        mesh=plsc.VectorSubcoreMesh(**_SC_MESH),
        out_type=jax.ShapeDtypeStruct((2 * NPAD, 128), jnp.float32),
        scratch_types=[
            pltpu.VMEM((CH,), jnp.int32),
            pltpu.VMEM((CH, 128), jnp.float32),
            pltpu.VMEM_SHARED((NPAD, 128), jnp.float32),
        ],
    )
    def deg_kernel(col_hbm, out_hbm, idxc, vbuf, deg_sh):
        cid = lax.axis_index("c")
        sid = lax.axis_index("s")
        _zero_buf(vbuf, 128)
        _zero_acc(vbuf, deg_sh, sid)

        def orow(r, _):
            for j in range(128 // 16):
                vbuf[r, pl.ds(j * 16, 16)] = jnp.full((16,), 1.0, jnp.float32)
            return 0

        lax.fori_loop(0, CH, orow, 0)
        plsc.subcore_barrier()

        ebase = cid * (EPAD // 2) + sid * ept

        def chunk(g, _):
            pltpu.sync_copy(col_hbm.at[pl.ds(ebase + g * CH, CH)], idxc)
            pltpu.sync_copy(vbuf, deg_sh.at[idxc], add=True)
            return 0

        lax.fori_loop(0, nchunk, chunk, 0)
        plsc.subcore_barrier()
        base_r = sid * RPT
        pltpu.sync_copy(
            deg_sh.at[pl.ds(base_r, RPT)],
            out_hbm.at[pl.ds(cid * NPAD + base_r, RPT)],
        )

    return deg_kernel


def _make_agg_kernel(edge_split):
    ept = EPAD // NTILE // (2 if edge_split else 1)
    nchunk = ept // CH

    @functools.partial(
        pl.kernel,
        mesh=plsc.VectorSubcoreMesh(**_SC_MESH),
        out_type=jax.ShapeDtypeStruct((2 * NPAD, 128), jnp.float32),
        scratch_types=[
            pltpu.VMEM((CH,), jnp.int32),
            pltpu.VMEM((CH,), jnp.int32),
            pltpu.VMEM((CH, 128), jnp.float32),
            pltpu.VMEM_SHARED((NPAD, 128), jnp.float32),
            pltpu.SemaphoreType.DMA,
        ],
    )
    def agg_kernel(row_hbm, col_hbm, y_hbm, out_hbm, idxr, idxc, buf, acc_sh, sem):
        cid = lax.axis_index("c")
        sid = lax.axis_index("s")
        _zero_buf(buf, 128)
        _zero_acc(buf, acc_sh, sid)
        plsc.subcore_barrier()

        if edge_split:
            ebase = cid * (EPAD // 2) + sid * ept
            noff = 0
        else:
            ebase = sid * ept
            noff = cid * NPAD

        def chunk(g, _):
            b = ebase + g * CH
            pltpu.sync_copy(row_hbm.at[pl.ds(b, CH)], idxr)
            pltpu.sync_copy(col_hbm.at[pl.ds(b, CH)], idxc)
            if not edge_split:
                for j in range(CH // 16):
                    idxr[pl.ds(j * 16, 16)] = idxr[pl.ds(j * 16, 16)] + noff
            pltpu.async_copy(y_hbm.at[idxr], buf, sem).wait()
            pltpu.sync_copy(buf, acc_sh.at[idxc], add=True)
            return 0

        lax.fori_loop(0, nchunk, chunk, 0)
        plsc.subcore_barrier()
        base_r = sid * RPT
        pltpu.sync_copy(
            acc_sh.at[pl.ds(base_r, RPT)],
            out_hbm.at[pl.ds(cid * NPAD + base_r, RPT)],
        )

    return agg_kernel


def _prep_body(degc_ref, xf_ref, dis_ref, y_ref):
    deg = degc_ref[0] + degc_ref[1] + 1.0
    dis = lax.rsqrt(deg)
    dis_ref[...] = dis
    y_ref[...] = dis * xf_ref[...]


def _prep_tc(degc, xfp):
    return pl.pallas_call(
        _prep_body,
        out_shape=(
            jax.ShapeDtypeStruct((NPAD, 1), jnp.float32),
            jax.ShapeDtypeStruct((NPAD, IN), jnp.float32),
        ),
    )(degc, xfp)


def _layer_body(acc_ref, yp_ref, dis_ref, lab_ref, bat_ref, W_ref, b_ref,
                yn_ref, pooled_ref, *, split, last):
    i = pl.program_id(0)
    dis = dis_ref[...]
    if split:
        agg = jnp.concatenate(
            [acc_ref[0] + yp_ref[0], acc_ref[1] + yp_ref[1]], axis=1) * dis
    else:
        agg = (acc_ref[0] + acc_ref[1] + yp_ref[...]) * dis
    lab = lab_ref[0]
    x2 = jnp.zeros((BN, H), jnp.float32)
    for c in range(C):
        v = jnp.dot(agg, W_ref[c], preferred_element_type=jnp.float32) + b_ref[c]
        x2 = jnp.where(lab == c, v, x2)
    bat = bat_ref[0]
    oh = (lax.broadcasted_iota(jnp.int32, (G, BN), 0) == bat
          ).astype(jnp.float32)

    @pl.when(i == 0)
    def _():
        pooled_ref[...] = jnp.zeros((G, H), jnp.float32)

    pooled_ref[...] += jnp.dot(oh, x2, preferred_element_type=jnp.float32)
    if not last:
        yn = x2 * dis
        yn_ref[0] = yn[:, : H // 2]
        yn_ref[1] = yn[:, H // 2 :]


def _common_specs(W, b3, split):
    yp_spec = (pl.BlockSpec((2, BN, 128), lambda i: (0, i, 0)) if split
               else pl.BlockSpec((BN, 128), lambda i: (i, 0)))
    return [
        pl.BlockSpec((2, BN, 128), lambda i: (0, i, 0)),
        yp_spec,
        pl.BlockSpec((BN, 1), lambda i: (i, 0)),
        pl.BlockSpec((1, BN, 1), lambda i: (i, 0, 0)),
        pl.BlockSpec((1, 1, BN), lambda i: (i, 0, 0)),
        pl.BlockSpec(W.shape, lambda i: (0, 0, 0)),
        pl.BlockSpec(b3.shape, lambda i: (0, 0, 0)),
    ]


def _layer_tc(acc, yp, dis, lab3, bat3, W, b3, split):
    out_specs = (
        pl.BlockSpec((2, BN, H // 2), lambda i: (0, i, 0)),
        pl.BlockSpec((G, H), lambda i: (0, 0)),
    )
    out_shape = (
        jax.ShapeDtypeStruct((2, NPAD, H // 2), jnp.float32),
        jax.ShapeDtypeStruct((G, H), jnp.float32),
    )
    return pl.pallas_call(
        functools.partial(_layer_body, split=split, last=False),
        grid=(NB,), in_specs=_common_specs(W, b3, split),
        out_specs=out_specs, out_shape=out_shape,
    )(acc, yp, dis, lab3, bat3, W, b3)


def _last_body(acc_ref, yp_ref, dis_ref, lab_ref, bat_ref, W_ref, b_ref,
               p0_ref, p1_ref, M1_ref, mb1_ref, M2_ref, mb2_ref,
               out_ref, pooled_ref):
    _layer_body(acc_ref, yp_ref, dis_ref, lab_ref, bat_ref, W_ref, b_ref,
                None, pooled_ref, split=True, last=True)
    i = pl.program_id(0)

    @pl.when(i == NB - 1)
    def _():
        h = jnp.concatenate([p0_ref[...], p1_ref[...], pooled_ref[...]], axis=1)
        hh = jnp.maximum(
            jnp.dot(h, M1_ref[...], preferred_element_type=jnp.float32)
            + mb1_ref[...], 0.0)
        out_ref[...] = (jnp.dot(hh, M2_ref[...], preferred_element_type=jnp.float32)
                        + mb2_ref[...])


def _last_tc(acc, yp, dis, lab3, bat3, W, b3, p0, p1, M1, mb1, M2, mb2):
    in_specs = _common_specs(W, b3, True) + [
        pl.BlockSpec((G, H), lambda i: (0, 0)),
        pl.BlockSpec((G, H), lambda i: (0, 0)),
        pl.BlockSpec(M1.shape, lambda i: (0, 0)),
        pl.BlockSpec((1, H), lambda i: (0, 0)),
        pl.BlockSpec(M2.shape, lambda i: (0, 0)),
        pl.BlockSpec((1, OUT), lambda i: (0, 0)),
    ]
    out_specs = (
        pl.BlockSpec((G, OUT), lambda i: (0, 0)),
        pl.BlockSpec((G, H), lambda i: (0, 0)),
    )
    out_shape = (
        jax.ShapeDtypeStruct((G, OUT), jnp.float32),
        jax.ShapeDtypeStruct((G, H), jnp.float32),
    )
    out, _ = pl.pallas_call(
        _last_body,
        grid=(NB,), in_specs=in_specs, out_specs=out_specs, out_shape=out_shape,
    )(acc, yp, dis, lab3, bat3, W, b3, p0, p1, M1, mb1, M2, mb2)
    return out


_deg_kernel = _make_deg_kernel()
_agg_e = _make_agg_kernel(edge_split=True)
_agg_f = _make_agg_kernel(edge_split=False)


def kernel(x_feat, clustering_labels, edge_index, batch,
           W0, b0, W1, b1, W2, b2, M1, mb1, M2, mb2):
    row = edge_index[0].astype(jnp.int32)
    col = edge_index[1].astype(jnp.int32)
    padi = (jnp.arange(EPAD - E, dtype=jnp.int32) % (NPAD - N)) + N
    rowp = jnp.concatenate([row, padi])
    colp = jnp.concatenate([col, padi])
    xfp = jnp.pad(x_feat, ((0, NPAD - N), (0, 0)))
    lab3 = jnp.pad(clustering_labels.astype(jnp.int32),
                   (0, NPAD - N)).reshape(NB, BN, 1)
    bat3 = jnp.pad(batch.astype(jnp.int32), (0, NPAD - N),
                   constant_values=-1).reshape(NB, 1, BN)

    degc = _deg_kernel(colp).reshape(2, NPAD, 128)[:, :, 0:1]
    dis, y0 = _prep_tc(degc, xfp)

    acc0 = _agg_e(rowp, colp, y0).reshape(2, NPAD, 128)
    y1, p0 = _layer_tc(acc0, y0, dis, lab3, bat3,
                       W0, b0.reshape(C, 1, H), split=False)
    acc1 = _agg_f(rowp, colp, y1.reshape(2 * NPAD, H // 2)).reshape(2, NPAD, 128)
    y2, p1 = _layer_tc(acc1, y1, dis, lab3, bat3,
                       W1, b1.reshape(C, 1, H), split=True)
    acc2 = _agg_f(rowp, colp, y2.reshape(2 * NPAD, H // 2)).reshape(2, NPAD, 128)
    out = _last_tc(acc2, y2, dis, lab3, bat3,
                   W2, b2.reshape(C, 1, H), p0, p1,
                   M1, mb1.reshape(1, H), M2, mb2.reshape(1, OUT))
    return out


# speedup vs baseline: 90.9806x; 1.5239x over previous
import functools

import jax
import jax.numpy as jnp
from jax import lax
from jax.experimental import pallas as pl
from jax.experimental.pallas import tpu as pltpu
from jax.experimental.pallas import tpu_sc as plsc

N = 10000
E = 320000
C = 8
IN = 128
H = 256
OUT = 128
G = 16

NPAD = 10240
EPAD = 327680
BN = 512
NB = NPAD // BN
CH = 128
NTILE = 16
RPT = NPAD // NTILE

_SC_MESH = dict(core_axis_name="c", subcore_axis_name="s")


def _zero_buf(buf, d):
    def zrow(r, _):
        for j in range(d // 16):
            buf[r, pl.ds(j * 16, 16)] = jnp.zeros((16,), jnp.float32)
        return 0

    lax.fori_loop(0, CH, zrow, 0)


def _zero_acc(buf, acc_sh, sid):
    base_r = sid * RPT
    for j in range(RPT // CH):
        pltpu.sync_copy(buf, acc_sh.at[pl.ds(base_r + j * CH, CH)])


def _make_deg_kernel():
    ept = EPAD // 2 // NTILE
    nchunk = ept // CH

    @functools.partial(
        pl.kernel,
        mesh=plsc.VectorSubcoreMesh(**_SC_MESH),
        out_type=jax.ShapeDtypeStruct((2 * NPAD, 128), jnp.float32),
        scratch_types=[
            pltpu.VMEM((CH,), jnp.int32),
            pltpu.VMEM((CH, 128), jnp.float32),
            pltpu.VMEM_SHARED((NPAD, 128), jnp.float32),
        ],
    )
    def deg_kernel(col_hbm, out_hbm, idxc, vbuf, deg_sh):
        cid = lax.axis_index("c")
        sid = lax.axis_index("s")
        _zero_buf(vbuf, 128)
        _zero_acc(vbuf, deg_sh, sid)

        def orow(r, _):
            for j in range(128 // 16):
                vbuf[r, pl.ds(j * 16, 16)] = jnp.full((16,), 1.0, jnp.float32)
            return 0

        lax.fori_loop(0, CH, orow, 0)
        plsc.subcore_barrier()

        ebase = cid * (EPAD // 2) + sid * ept

        def chunk(g, _):
            pltpu.sync_copy(col_hbm.at[pl.ds(ebase + g * CH, CH)], idxc)
            pltpu.sync_copy(vbuf, deg_sh.at[idxc], add=True)
            return 0

        lax.fori_loop(0, nchunk, chunk, 0)
        plsc.subcore_barrier()
        base_r = sid * RPT
        pltpu.sync_copy(
            deg_sh.at[pl.ds(base_r, RPT)],
            out_hbm.at[pl.ds(cid * NPAD + base_r, RPT)],
        )

    return deg_kernel


def _make_agg_kernel(edge_split):
    ept = EPAD // NTILE // (2 if edge_split else 1)
    nchunk = ept // CH

    @functools.partial(
        pl.kernel,
        mesh=plsc.VectorSubcoreMesh(**_SC_MESH),
        out_type=jax.ShapeDtypeStruct((2 * NPAD, 128), jnp.float32),
        scratch_types=[
            pltpu.VMEM((CH,), jnp.int32),
            pltpu.VMEM((CH,), jnp.int32),
            pltpu.VMEM((CH,), jnp.int32),
            pltpu.VMEM((CH,), jnp.int32),
            pltpu.VMEM((CH, 128), jnp.float32),
            pltpu.VMEM((CH, 128), jnp.float32),
            pltpu.VMEM_SHARED((NPAD, 128), jnp.float32),
            pltpu.SemaphoreType.DMA,
            pltpu.SemaphoreType.DMA,
        ],
    )
    def agg_kernel(row_hbm, col_hbm, y_hbm, out_hbm,
                   idxr0, idxr1, idxc0, idxc1, buf0, buf1, acc_sh,
                   sem0, sem1):
        cid = lax.axis_index("c")
        sid = lax.axis_index("s")
        _zero_buf(buf0, 128)
        _zero_acc(buf0, acc_sh, sid)
        plsc.subcore_barrier()

        if edge_split:
            ebase = cid * (EPAD // 2) + sid * ept
            noff = 0
        else:
            ebase = sid * ept
            noff = cid * NPAD

        def load_idx(g, idxr, idxc):
            b = ebase + g * CH
            pltpu.sync_copy(row_hbm.at[pl.ds(b, CH)], idxr)
            pltpu.sync_copy(col_hbm.at[pl.ds(b, CH)], idxc)
            if not edge_split:
                for j in range(CH // 16):
                    idxr[pl.ds(j * 16, 16)] = idxr[pl.ds(j * 16, 16)] + noff

        def gather(idxr, buf, sem):
            return pltpu.make_async_copy(y_hbm.at[idxr], buf, sem)

        load_idx(0, idxr0, idxc0)
        gather(idxr0, buf0, sem0).start()

        def pair(i, _):
            g0 = 2 * i
            load_idx(g0 + 1, idxr1, idxc1)
            gather(idxr1, buf1, sem1).start()
            gather(idxr0, buf0, sem0).wait()
            pltpu.sync_copy(buf0, acc_sh.at[idxc0], add=True)

            @pl.when(g0 + 2 < nchunk)
            def _():
                load_idx(g0 + 2, idxr0, idxc0)
                gather(idxr0, buf0, sem0).start()

            gather(idxr1, buf1, sem1).wait()
            pltpu.sync_copy(buf1, acc_sh.at[idxc1], add=True)
            return 0

        lax.fori_loop(0, nchunk // 2, pair, 0)
        plsc.subcore_barrier()
        base_r = sid * RPT
        pltpu.sync_copy(
            acc_sh.at[pl.ds(base_r, RPT)],
            out_hbm.at[pl.ds(cid * NPAD + base_r, RPT)],
        )

    return agg_kernel


def _prep_body(degc_ref, xf_ref, dis_ref, y_ref):
    deg = degc_ref[0] + degc_ref[1] + 1.0
    dis = lax.rsqrt(deg)
    dis_ref[...] = dis
    y_ref[...] = dis * xf_ref[...]


def _prep_tc(degc, xfp):
    return pl.pallas_call(
        _prep_body,
        out_shape=(
            jax.ShapeDtypeStruct((NPAD, 1), jnp.float32),
            jax.ShapeDtypeStruct((NPAD, IN), jnp.float32),
        ),
    )(degc, xfp)


def _layer_body(acc_ref, yp_ref, dis_ref, lab_ref, bat_ref, W_ref, b_ref,
                yn_ref, pooled_ref, *, split, last):
    i = pl.program_id(0)
    dis = dis_ref[...]
    if split:
        agg = jnp.concatenate(
            [acc_ref[0] + yp_ref[0], acc_ref[1] + yp_ref[1]], axis=1) * dis
    else:
        agg = (acc_ref[0] + acc_ref[1] + yp_ref[...]) * dis
    lab = lab_ref[0]
    x2 = jnp.zeros((BN, H), jnp.float32)
    for c in range(C):
        v = jnp.dot(agg, W_ref[c], preferred_element_type=jnp.float32) + b_ref[c]
        x2 = jnp.where(lab == c, v, x2)
    bat = bat_ref[0]
    oh = (lax.broadcasted_iota(jnp.int32, (G, BN), 0) == bat
          ).astype(jnp.float32)

    @pl.when(i == 0)
    def _():
        pooled_ref[...] = jnp.zeros((G, H), jnp.float32)

    pooled_ref[...] += jnp.dot(oh, x2, preferred_element_type=jnp.float32)
    if not last:
        yn = x2 * dis
        yn_ref[0] = yn[:, : H // 2]
        yn_ref[1] = yn[:, H // 2 :]


def _common_specs(W, b3, split):
    yp_spec = (pl.BlockSpec((2, BN, 128), lambda i: (0, i, 0)) if split
               else pl.BlockSpec((BN, 128), lambda i: (i, 0)))
    return [
        pl.BlockSpec((2, BN, 128), lambda i: (0, i, 0)),
        yp_spec,
        pl.BlockSpec((BN, 1), lambda i: (i, 0)),
        pl.BlockSpec((1, BN, 1), lambda i: (i, 0, 0)),
        pl.BlockSpec((1, 1, BN), lambda i: (i, 0, 0)),
        pl.BlockSpec(W.shape, lambda i: (0, 0, 0)),
        pl.BlockSpec(b3.shape, lambda i: (0, 0, 0)),
    ]


def _layer_tc(acc, yp, dis, lab3, bat3, W, b3, split):
    out_specs = (
        pl.BlockSpec((2, BN, H // 2), lambda i: (0, i, 0)),
        pl.BlockSpec((G, H), lambda i: (0, 0)),
    )
    out_shape = (
        jax.ShapeDtypeStruct((2, NPAD, H // 2), jnp.float32),
        jax.ShapeDtypeStruct((G, H), jnp.float32),
    )
    return pl.pallas_call(
        functools.partial(_layer_body, split=split, last=False),
        grid=(NB,), in_specs=_common_specs(W, b3, split),
        out_specs=out_specs, out_shape=out_shape,
    )(acc, yp, dis, lab3, bat3, W, b3)


def _last_body(acc_ref, yp_ref, dis_ref, lab_ref, bat_ref, W_ref, b_ref,
               p0_ref, p1_ref, M1_ref, mb1_ref, M2_ref, mb2_ref,
               out_ref, pooled_ref):
    _layer_body(acc_ref, yp_ref, dis_ref, lab_ref, bat_ref, W_ref, b_ref,
                None, pooled_ref, split=True, last=True)
    i = pl.program_id(0)

    @pl.when(i == NB - 1)
    def _():
        h = jnp.concatenate([p0_ref[...], p1_ref[...], pooled_ref[...]], axis=1)
        hh = jnp.maximum(
            jnp.dot(h, M1_ref[...], preferred_element_type=jnp.float32)
            + mb1_ref[...], 0.0)
        out_ref[...] = (jnp.dot(hh, M2_ref[...], preferred_element_type=jnp.float32)
                        + mb2_ref[...])


def _last_tc(acc, yp, dis, lab3, bat3, W, b3, p0, p1, M1, mb1, M2, mb2):
    in_specs = _common_specs(W, b3, True) + [
        pl.BlockSpec((G, H), lambda i: (0, 0)),
        pl.BlockSpec((G, H), lambda i: (0, 0)),
        pl.BlockSpec(M1.shape, lambda i: (0, 0)),
        pl.BlockSpec((1, H), lambda i: (0, 0)),
        pl.BlockSpec(M2.shape, lambda i: (0, 0)),
        pl.BlockSpec((1, OUT), lambda i: (0, 0)),
    ]
    out_specs = (
        pl.BlockSpec((G, OUT), lambda i: (0, 0)),
        pl.BlockSpec((G, H), lambda i: (0, 0)),
    )
    out_shape = (
        jax.ShapeDtypeStruct((G, OUT), jnp.float32),
        jax.ShapeDtypeStruct((G, H), jnp.float32),
    )
    out, _ = pl.pallas_call(
        _last_body,
        grid=(NB,), in_specs=in_specs, out_specs=out_specs, out_shape=out_shape,
    )(acc, yp, dis, lab3, bat3, W, b3, p0, p1, M1, mb1, M2, mb2)
    return out


_deg_kernel = _make_deg_kernel()
_agg_e = _make_agg_kernel(edge_split=True)
_agg_f = _make_agg_kernel(edge_split=False)


def kernel(x_feat, clustering_labels, edge_index, batch,
           W0, b0, W1, b1, W2, b2, M1, mb1, M2, mb2):
    row = edge_index[0].astype(jnp.int32)
    col = edge_index[1].astype(jnp.int32)
    padi = (jnp.arange(EPAD - E, dtype=jnp.int32) % (NPAD - N)) + N
    rowp = jnp.concatenate([row, padi])
    colp = jnp.concatenate([col, padi])
    xfp = jnp.pad(x_feat, ((0, NPAD - N), (0, 0)))
    lab3 = jnp.pad(clustering_labels.astype(jnp.int32),
                   (0, NPAD - N)).reshape(NB, BN, 1)
    bat3 = jnp.pad(batch.astype(jnp.int32), (0, NPAD - N),
                   constant_values=-1).reshape(NB, 1, BN)

    degc = _deg_kernel(colp).reshape(2, NPAD, 128)[:, :, 0:1]
    dis, y0 = _prep_tc(degc, xfp)

    acc0 = _agg_e(rowp, colp, y0).reshape(2, NPAD, 128)
    y1, p0 = _layer_tc(acc0, y0, dis, lab3, bat3,
                       W0, b0.reshape(C, 1, H), split=False)
    acc1 = _agg_f(rowp, colp, y1.reshape(2 * NPAD, H // 2)).reshape(2, NPAD, 128)
    y2, p1 = _layer_tc(acc1, y1, dis, lab3, bat3,
                       W1, b1.reshape(C, 1, H), split=True)
    acc2 = _agg_f(rowp, colp, y2.reshape(2 * NPAD, H // 2)).reshape(2, NPAD, 128)
    out = _last_tc(acc2, y2, dis, lab3, bat3,
                   W2, b2.reshape(C, 1, H), p0, p1,
                   M1, mb1.reshape(1, H), M2, mb2.reshape(1, OUT))
    return out


# speedup vs baseline: 124.7306x; 1.3710x over previous
import functools

import jax
import jax.numpy as jnp
from jax import lax
from jax.experimental import pallas as pl
from jax.experimental.pallas import tpu as pltpu
from jax.experimental.pallas import tpu_sc as plsc

N = 10000
E = 320000
C = 8
IN = 128
H = 256
OUT = 128
G = 16

NPAD = 10240
EPAD = 327680
BN = 512
NB = NPAD // BN
CH = 128
NTILE = 16
RPT = NPAD // NTILE

_SC_MESH = dict(core_axis_name="c", subcore_axis_name="s")


def _zero_buf(buf, d):
    def zrow(r, _):
        for j in range(d // 16):
            buf[r, pl.ds(j * 16, 16)] = jnp.zeros((16,), jnp.float32)
        return 0

    lax.fori_loop(0, CH, zrow, 0)


def _zero_acc(buf, acc_sh, sid):
    base_r = sid * RPT
    for j in range(RPT // CH):
        pltpu.sync_copy(buf, acc_sh.at[pl.ds(base_r + j * CH, CH)])


def _make_deg_kernel():
    ept = EPAD // 2 // NTILE
    nchunk = ept // CH

    @functools.partial(
        pl.kernel,
        mesh=plsc.VectorSubcoreMesh(**_SC_MESH),
        out_type=jax.ShapeDtypeStruct((2 * NPAD, 128), jnp.float32),
        scratch_types=[
            pltpu.VMEM((EPAD // 2 // NTILE,), jnp.int32),
            pltpu.VMEM((CH, 128), jnp.float32),
            pltpu.VMEM_SHARED((NPAD, 128), jnp.float32),
        ],
    )
    def deg_kernel(col_hbm, out_hbm, idxc, vbuf, deg_sh):
        cid = lax.axis_index("c")
        sid = lax.axis_index("s")
        _zero_buf(vbuf, 128)
        _zero_acc(vbuf, deg_sh, sid)

        def orow(r, _):
            for j in range(128 // 16):
                vbuf[r, pl.ds(j * 16, 16)] = jnp.full((16,), 1.0, jnp.float32)
            return 0

        lax.fori_loop(0, CH, orow, 0)
        ebase = cid * (EPAD // 2) + sid * ept
        pltpu.sync_copy(col_hbm.at[pl.ds(ebase, ept)], idxc)
        plsc.subcore_barrier()

        def chunk(g, _):
            pltpu.sync_copy(vbuf, deg_sh.at[idxc.at[pl.ds(g * CH, CH)]],
                            add=True)
            return 0

        lax.fori_loop(0, nchunk, chunk, 0)
        plsc.subcore_barrier()
        base_r = sid * RPT
        pltpu.sync_copy(
            deg_sh.at[pl.ds(base_r, RPT)],
            out_hbm.at[pl.ds(cid * NPAD + base_r, RPT)],
        )

    return deg_kernel


def _make_agg_kernel(edge_split):
    ept = EPAD // NTILE // (2 if edge_split else 1)
    nchunk = ept // CH

    @functools.partial(
        pl.kernel,
        mesh=plsc.VectorSubcoreMesh(**_SC_MESH),
        out_type=jax.ShapeDtypeStruct((2 * NPAD, 128), jnp.float32),
        scratch_types=(
            [pltpu.VMEM((CH,), jnp.int32)] * 8
            + [pltpu.VMEM((CH, 128), jnp.float32)] * 2
            + [pltpu.VMEM_SHARED((NPAD, 128), jnp.float32)]
            + [pltpu.SemaphoreType.DMA] * 10
        ),
    )
    def agg_kernel(row_hbm, col_hbm, y_hbm, out_hbm,
                   ir0, ir1, ir2, ir3, ic0, ic1, ic2, ic3, buf0, buf1,
                   acc_sh, g0s, g1s, r0s, r1s, r2s, r3s, c0s, c1s, c2s, c3s):
        cid = lax.axis_index("c")
        sid = lax.axis_index("s")
        _zero_buf(buf0, 128)
        _zero_acc(buf0, acc_sh, sid)
        plsc.subcore_barrier()

        if edge_split:
            ebase = cid * (EPAD // 2) + sid * ept
            noff = 0
        else:
            ebase = sid * ept
            noff = cid * NPAD

        idxr = (ir0, ir1, ir2, ir3)
        idxc = (ic0, ic1, ic2, ic3)
        buf = (buf0, buf1)
        gsem = (g0s, g1s)
        rsem = (r0s, r1s, r2s, r3s)
        csem = (c0s, c1s, c2s, c3s)

        def idx_load(g, s):
            b = ebase + g * CH
            return (pltpu.make_async_copy(row_hbm.at[pl.ds(b, CH)],
                                          idxr[s], rsem[s]),
                    pltpu.make_async_copy(col_hbm.at[pl.ds(b, CH)],
                                          idxc[s], csem[s]))

        def idx_start(g, s):
            a, b = idx_load(g, s)
            a.start()
            b.start()

        def idx_wait_shift(g, s):
            a, b = idx_load(g, s)
            a.wait()
            b.wait()
            if not edge_split:
                for j in range(CH // 16):
                    idxr[s][pl.ds(j * 16, 16)] = (
                        idxr[s][pl.ds(j * 16, 16)] + noff)

        def gather(s, bs):
            return pltpu.make_async_copy(y_hbm.at[idxr[s]], buf[bs], gsem[bs])

        idx_wait_shift_0 = idx_load(0, 0)
        idx_wait_shift_0[0].start()
        idx_wait_shift_0[1].start()
        idx_wait_shift(0, 0)
        gather(0, 0).start()
        idx_start(1, 1)

        def quad(i, _):
            q = 4 * i
            for k in range(4):
                g = q + k
                is_ = k
                bs = k % 2

                def stage_a(g=g, s=(k + 2) % 4):
                    idx_start(g + 2, s)

                def stage_b(g=g, s=(k + 1) % 4, nb=(k + 1) % 2):
                    idx_wait_shift(g + 1, s)
                    gather(s, nb).start()

                if k < 2:
                    stage_a()
                else:
                    pl.when(g + 2 < nchunk)(stage_a)
                if k < 3:
                    stage_b()
                else:
                    pl.when(g + 1 < nchunk)(stage_b)
                gather(is_, bs).wait()
                pltpu.sync_copy(buf[bs], acc_sh.at[idxc[is_]], add=True)
            return 0

        lax.fori_loop(0, nchunk // 4, quad, 0)
        plsc.subcore_barrier()
        base_r = sid * RPT
        pltpu.sync_copy(
            acc_sh.at[pl.ds(base_r, RPT)],
            out_hbm.at[pl.ds(cid * NPAD + base_r, RPT)],
        )

    return agg_kernel


def _prep_body(degc_ref, xf_ref, dis_ref, y_ref):
    deg = degc_ref[0] + degc_ref[1] + 1.0
    dis = lax.rsqrt(deg)
    dis_ref[...] = dis
    y_ref[...] = dis * xf_ref[...]


def _prep_tc(degc, xfp):
    return pl.pallas_call(
        _prep_body,
        out_shape=(
            jax.ShapeDtypeStruct((NPAD, 1), jnp.float32),
            jax.ShapeDtypeStruct((NPAD, IN), jnp.float32),
        ),
    )(degc, xfp)


def _layer_body(acc_ref, yp_ref, dis_ref, lab_ref, bat_ref, W_ref, b_ref,
                yn_ref, pooled_ref, *, split, last):
    i = pl.program_id(0)
    dis = dis_ref[...]
    if split:
        agg = jnp.concatenate(
            [acc_ref[0] + yp_ref[0], acc_ref[1] + yp_ref[1]], axis=1) * dis
    else:
        agg = (acc_ref[0] + acc_ref[1] + yp_ref[...]) * dis
    lab = lab_ref[0]
    x2 = jnp.zeros((BN, H), jnp.float32)
    for c in range(C):
        v = jnp.dot(agg, W_ref[c], preferred_element_type=jnp.float32) + b_ref[c]
        x2 = jnp.where(lab == c, v, x2)
    bat = bat_ref[0]
    oh = (lax.broadcasted_iota(jnp.int32, (G, BN), 0) == bat
          ).astype(jnp.float32)

    @pl.when(i == 0)
    def _():
        pooled_ref[...] = jnp.zeros((G, H), jnp.float32)

    pooled_ref[...] += jnp.dot(oh, x2, preferred_element_type=jnp.float32)
    if not last:
        yn = x2 * dis
        yn_ref[0] = yn[:, : H // 2]
        yn_ref[1] = yn[:, H // 2 :]


def _common_specs(W, b3, split):
    yp_spec = (pl.BlockSpec((2, BN, 128), lambda i: (0, i, 0)) if split
               else pl.BlockSpec((BN, 128), lambda i: (i, 0)))
    return [
        pl.BlockSpec((2, BN, 128), lambda i: (0, i, 0)),
        yp_spec,
        pl.BlockSpec((BN, 1), lambda i: (i, 0)),
        pl.BlockSpec((1, BN, 1), lambda i: (i, 0, 0)),
        pl.BlockSpec((1, 1, BN), lambda i: (i, 0, 0)),
        pl.BlockSpec(W.shape, lambda i: (0, 0, 0)),
        pl.BlockSpec(b3.shape, lambda i: (0, 0, 0)),
    ]


def _layer_tc(acc, yp, dis, lab3, bat3, W, b3, split):
    out_specs = (
        pl.BlockSpec((2, BN, H // 2), lambda i: (0, i, 0)),
        pl.BlockSpec((G, H), lambda i: (0, 0)),
    )
    out_shape = (
        jax.ShapeDtypeStruct((2, NPAD, H // 2), jnp.float32),
        jax.ShapeDtypeStruct((G, H), jnp.float32),
    )
    return pl.pallas_call(
        functools.partial(_layer_body, split=split, last=False),
        grid=(NB,), in_specs=_common_specs(W, b3, split),
        out_specs=out_specs, out_shape=out_shape,
    )(acc, yp, dis, lab3, bat3, W, b3)


def _last_body(acc_ref, yp_ref, dis_ref, lab_ref, bat_ref, W_ref, b_ref,
               p0_ref, p1_ref, M1_ref, mb1_ref, M2_ref, mb2_ref,
               out_ref, pooled_ref):
    _layer_body(acc_ref, yp_ref, dis_ref, lab_ref, bat_ref, W_ref, b_ref,
                None, pooled_ref, split=True, last=True)
    i = pl.program_id(0)

    @pl.when(i == NB - 1)
    def _():
        h = jnp.concatenate([p0_ref[...], p1_ref[...], pooled_ref[...]], axis=1)
        hh = jnp.maximum(
            jnp.dot(h, M1_ref[...], preferred_element_type=jnp.float32)
            + mb1_ref[...], 0.0)
        out_ref[...] = (jnp.dot(hh, M2_ref[...], preferred_element_type=jnp.float32)
                        + mb2_ref[...])


def _last_tc(acc, yp, dis, lab3, bat3, W, b3, p0, p1, M1, mb1, M2, mb2):
    in_specs = _common_specs(W, b3, True) + [
        pl.BlockSpec((G, H), lambda i: (0, 0)),
        pl.BlockSpec((G, H), lambda i: (0, 0)),
        pl.BlockSpec(M1.shape, lambda i: (0, 0)),
        pl.BlockSpec((1, H), lambda i: (0, 0)),
        pl.BlockSpec(M2.shape, lambda i: (0, 0)),
        pl.BlockSpec((1, OUT), lambda i: (0, 0)),
    ]
    out_specs = (
        pl.BlockSpec((G, OUT), lambda i: (0, 0)),
        pl.BlockSpec((G, H), lambda i: (0, 0)),
    )
    out_shape = (
        jax.ShapeDtypeStruct((G, OUT), jnp.float32),
        jax.ShapeDtypeStruct((G, H), jnp.float32),
    )
    out, _ = pl.pallas_call(
        _last_body,
        grid=(NB,), in_specs=in_specs, out_specs=out_specs, out_shape=out_shape,
    )(acc, yp, dis, lab3, bat3, W, b3, p0, p1, M1, mb1, M2, mb2)
    return out


_deg_kernel = _make_deg_kernel()
_agg_e = _make_agg_kernel(edge_split=True)
_agg_f = _make_agg_kernel(edge_split=False)


def kernel(x_feat, clustering_labels, edge_index, batch,
           W0, b0, W1, b1, W2, b2, M1, mb1, M2, mb2):
    row = edge_index[0].astype(jnp.int32)
    col = edge_index[1].astype(jnp.int32)
    padi = (jnp.arange(EPAD - E, dtype=jnp.int32) % (NPAD - N)) + N
    rowp = jnp.concatenate([row, padi])
    colp = jnp.concatenate([col, padi])
    xfp = jnp.pad(x_feat, ((0, NPAD - N), (0, 0)))
    lab3 = jnp.pad(clustering_labels.astype(jnp.int32),
                   (0, NPAD - N)).reshape(NB, BN, 1)
    bat3 = jnp.pad(batch.astype(jnp.int32), (0, NPAD - N),
                   constant_values=-1).reshape(NB, 1, BN)

    degc = _deg_kernel(colp).reshape(2, NPAD, 128)[:, :, 0:1]
    dis, y0 = _prep_tc(degc, xfp)

    acc0 = _agg_e(rowp, colp, y0).reshape(2, NPAD, 128)
    y1, p0 = _layer_tc(acc0, y0, dis, lab3, bat3,
                       W0, b0.reshape(C, 1, H), split=False)
    acc1 = _agg_f(rowp, colp, y1.reshape(2 * NPAD, H // 2)).reshape(2, NPAD, 128)
    y2, p1 = _layer_tc(acc1, y1, dis, lab3, bat3,
                       W1, b1.reshape(C, 1, H), split=True)
    acc2 = _agg_f(rowp, colp, y2.reshape(2 * NPAD, H // 2)).reshape(2, NPAD, 128)
    out = _last_tc(acc2, y2, dis, lab3, bat3,
                   W2, b2.reshape(C, 1, H), p0, p1,
                   M1, mb1.reshape(1, H), M2, mb2.reshape(1, OUT))
    return out


# speedup vs baseline: 125.0241x; 1.0024x over previous
import functools

import jax
import jax.numpy as jnp
from jax import lax
from jax.experimental import pallas as pl
from jax.experimental.pallas import tpu as pltpu
from jax.experimental.pallas import tpu_sc as plsc

N = 10000
E = 320000
C = 8
IN = 128
H = 256
OUT = 128
G = 16

NPAD = 10240
EPAD = 327680
BN = 512
NB = NPAD // BN
CH = 128
NTILE = 16
RPT = NPAD // NTILE

_SC_MESH = dict(core_axis_name="c", subcore_axis_name="s")


def _zero_buf(buf, d):
    def zrow(r, _):
        for j in range(d // 16):
            buf[r, pl.ds(j * 16, 16)] = jnp.zeros((16,), jnp.float32)
        return 0

    lax.fori_loop(0, CH, zrow, 0)


def _zero_acc(buf, acc_sh, sid):
    base_r = sid * RPT
    for j in range(RPT // CH):
        pltpu.sync_copy(buf, acc_sh.at[pl.ds(base_r + j * CH, CH)])


def _make_deg_kernel():
    ept = EPAD // 2 // NTILE
    nchunk = ept // CH

    @functools.partial(
        pl.kernel,
        mesh=plsc.VectorSubcoreMesh(**_SC_MESH),
        out_type=jax.ShapeDtypeStruct((2 * NPAD, 128), jnp.float32),
        scratch_types=[
            pltpu.VMEM((EPAD // 2 // NTILE,), jnp.int32),
            pltpu.VMEM((CH, 128), jnp.float32),
            pltpu.VMEM_SHARED((NPAD, 128), jnp.float32),
        ],
    )
    def deg_kernel(col_hbm, out_hbm, idxc, vbuf, deg_sh):
        cid = lax.axis_index("c")
        sid = lax.axis_index("s")
        _zero_buf(vbuf, 128)
        _zero_acc(vbuf, deg_sh, sid)

        def orow(r, _):
            for j in range(128 // 16):
                vbuf[r, pl.ds(j * 16, 16)] = jnp.full((16,), 1.0, jnp.float32)
            return 0

        lax.fori_loop(0, CH, orow, 0)
        ebase = cid * (EPAD // 2) + sid * ept
        pltpu.sync_copy(col_hbm.at[pl.ds(ebase, ept)], idxc)
        plsc.subcore_barrier()

        def chunk(g, _):
            pltpu.sync_copy(vbuf, deg_sh.at[idxc.at[pl.ds(g * CH, CH)]],
                            add=True)
            return 0

        lax.fori_loop(0, nchunk, chunk, 0)
        plsc.subcore_barrier()
        base_r = sid * RPT
        pltpu.sync_copy(
            deg_sh.at[pl.ds(base_r, RPT)],
            out_hbm.at[pl.ds(cid * NPAD + base_r, RPT)],
        )

    return deg_kernel


def _make_agg_kernel(edge_split):
    ept = EPAD // NTILE // (2 if edge_split else 1)
    nchunk = ept // CH

    @functools.partial(
        pl.kernel,
        mesh=plsc.VectorSubcoreMesh(**_SC_MESH),
        out_type=jax.ShapeDtypeStruct((2 * NPAD, 128), jnp.float32),
        scratch_types=(
            [pltpu.VMEM((CH,), jnp.int32)] * 8
            + [pltpu.VMEM((CH, 128), jnp.float32)] * 2
            + [pltpu.VMEM_SHARED((NPAD, 128), jnp.float32)]
            + [pltpu.SemaphoreType.DMA] * 10
        ),
    )
    def agg_kernel(row_hbm, col_hbm, y_hbm, out_hbm,
                   ir0, ir1, ir2, ir3, ic0, ic1, ic2, ic3, buf0, buf1,
                   acc_sh, g0s, g1s, r0s, r1s, r2s, r3s, c0s, c1s, c2s, c3s):
        cid = lax.axis_index("c")
        sid = lax.axis_index("s")
        _zero_buf(buf0, 128)
        _zero_acc(buf0, acc_sh, sid)
        plsc.subcore_barrier()

        if edge_split:
            ebase = cid * (EPAD // 2) + sid * ept
            noff = 0
        else:
            ebase = sid * ept
            noff = cid * NPAD

        idxr = (ir0, ir1, ir2, ir3)
        idxc = (ic0, ic1, ic2, ic3)
        buf = (buf0, buf1)
        gsem = (g0s, g1s)
        rsem = (r0s, r1s, r2s, r3s)
        csem = (c0s, c1s, c2s, c3s)

        def idx_load(g, s):
            b = ebase + g * CH
            return (pltpu.make_async_copy(row_hbm.at[pl.ds(b, CH)],
                                          idxr[s], rsem[s]),
                    pltpu.make_async_copy(col_hbm.at[pl.ds(b, CH)],
                                          idxc[s], csem[s]))

        def idx_start(g, s):
            a, b = idx_load(g, s)
            a.start()
            b.start()

        def idx_wait_shift(g, s):
            a, b = idx_load(g, s)
            a.wait()
            b.wait()
            if not edge_split:
                for j in range(CH // 16):
                    idxr[s][pl.ds(j * 16, 16)] = (
                        idxr[s][pl.ds(j * 16, 16)] + noff)

        def gather(s, bs):
            return pltpu.make_async_copy(y_hbm.at[idxr[s]], buf[bs], gsem[bs])

        idx_wait_shift_0 = idx_load(0, 0)
        idx_wait_shift_0[0].start()
        idx_wait_shift_0[1].start()
        idx_wait_shift(0, 0)
        gather(0, 0).start()
        idx_start(1, 1)

        def quad(i, _):
            q = 4 * i
            for k in range(4):
                g = q + k
                is_ = k
                bs = k % 2

                def stage_a(g=g, s=(k + 2) % 4):
                    idx_start(g + 2, s)

                def stage_b(g=g, s=(k + 1) % 4, nb=(k + 1) % 2):
                    idx_wait_shift(g + 1, s)
                    gather(s, nb).start()

                if k < 2:
                    stage_a()
                else:
                    pl.when(g + 2 < nchunk)(stage_a)
                if k < 3:
                    stage_b()
                else:
                    pl.when(g + 1 < nchunk)(stage_b)
                gather(is_, bs).wait()
                pltpu.sync_copy(buf[bs], acc_sh.at[idxc[is_]], add=True)
            return 0

        lax.fori_loop(0, nchunk // 4, quad, 0)
        plsc.subcore_barrier()
        base_r = sid * RPT
        pltpu.sync_copy(
            acc_sh.at[pl.ds(base_r, RPT)],
            out_hbm.at[pl.ds(cid * NPAD + base_r, RPT)],
        )

    return agg_kernel


def _prep_body(degc_ref, xf_ref, dis_ref, y_ref):
    deg = degc_ref[0] + degc_ref[1] + 1.0
    dis = lax.rsqrt(deg)
    dis_ref[...] = dis
    y_ref[...] = dis * xf_ref[...]


def _prep_tc(degc, xfp):
    return pl.pallas_call(
        _prep_body,
        out_shape=(
            jax.ShapeDtypeStruct((NPAD, 1), jnp.float32),
            jax.ShapeDtypeStruct((NPAD, IN), jnp.float32),
        ),
    )(degc, xfp)


def _layer_body(acc_ref, yp_ref, dis_ref, lab_ref, bat_ref, W_ref, b_ref,
                yn_ref, pooled_ref, *, split, last):
    i = pl.program_id(0)
    dis = dis_ref[...]
    if split:
        agg = jnp.concatenate(
            [acc_ref[0] + yp_ref[0], acc_ref[1] + yp_ref[1]], axis=1) * dis
    else:
        agg = (acc_ref[0] + acc_ref[1] + yp_ref[...]) * dis
    lab = lab_ref[0]
    aggh = agg.astype(jnp.bfloat16)
    x2 = jnp.zeros((BN, H), jnp.float32)
    for c in range(C):
        v = jnp.dot(aggh, W_ref[c].astype(jnp.bfloat16),
                    preferred_element_type=jnp.float32) + b_ref[c]
        x2 = jnp.where(lab == c, v, x2)
    bat = bat_ref[0]
    oh = (lax.broadcasted_iota(jnp.int32, (G, BN), 0) == bat
          ).astype(jnp.float32)

    @pl.when(i == 0)
    def _():
        pooled_ref[...] = jnp.zeros((G, H), jnp.float32)

    pooled_ref[...] += jnp.dot(oh, x2, preferred_element_type=jnp.float32)
    if not last:
        yn = x2 * dis
        yn_ref[0] = yn[:, : H // 2]
        yn_ref[1] = yn[:, H // 2 :]


def _common_specs(W, b3, split):
    yp_spec = (pl.BlockSpec((2, BN, 128), lambda i: (0, i, 0)) if split
               else pl.BlockSpec((BN, 128), lambda i: (i, 0)))
    return [
        pl.BlockSpec((2, BN, 128), lambda i: (0, i, 0)),
        yp_spec,
        pl.BlockSpec((BN, 1), lambda i: (i, 0)),
        pl.BlockSpec((1, BN, 1), lambda i: (i, 0, 0)),
        pl.BlockSpec((1, 1, BN), lambda i: (i, 0, 0)),
        pl.BlockSpec(W.shape, lambda i: (0, 0, 0)),
        pl.BlockSpec(b3.shape, lambda i: (0, 0, 0)),
    ]


def _layer_tc(acc, yp, dis, lab3, bat3, W, b3, split):
    out_specs = (
        pl.BlockSpec((2, BN, H // 2), lambda i: (0, i, 0)),
        pl.BlockSpec((G, H), lambda i: (0, 0)),
    )
    out_shape = (
        jax.ShapeDtypeStruct((2, NPAD, H // 2), jnp.float32),
        jax.ShapeDtypeStruct((G, H), jnp.float32),
    )
    return pl.pallas_call(
        functools.partial(_layer_body, split=split, last=False),
        grid=(NB,), in_specs=_common_specs(W, b3, split),
        out_specs=out_specs, out_shape=out_shape,
    )(acc, yp, dis, lab3, bat3, W, b3)


def _last_body(acc_ref, yp_ref, dis_ref, lab_ref, bat_ref, W_ref, b_ref,
               p0_ref, p1_ref, M1_ref, mb1_ref, M2_ref, mb2_ref,
               out_ref, pooled_ref):
    _layer_body(acc_ref, yp_ref, dis_ref, lab_ref, bat_ref, W_ref, b_ref,
                None, pooled_ref, split=True, last=True)
    i = pl.program_id(0)

    @pl.when(i == NB - 1)
    def _():
        h = jnp.concatenate([p0_ref[...], p1_ref[...], pooled_ref[...]], axis=1)
        hh = jnp.maximum(
            jnp.dot(h, M1_ref[...], preferred_element_type=jnp.float32)
            + mb1_ref[...], 0.0)
        out_ref[...] = (jnp.dot(hh, M2_ref[...], preferred_element_type=jnp.float32)
                        + mb2_ref[...])


def _last_tc(acc, yp, dis, lab3, bat3, W, b3, p0, p1, M1, mb1, M2, mb2):
    in_specs = _common_specs(W, b3, True) + [
        pl.BlockSpec((G, H), lambda i: (0, 0)),
        pl.BlockSpec((G, H), lambda i: (0, 0)),
        pl.BlockSpec(M1.shape, lambda i: (0, 0)),
        pl.BlockSpec((1, H), lambda i: (0, 0)),
        pl.BlockSpec(M2.shape, lambda i: (0, 0)),
        pl.BlockSpec((1, OUT), lambda i: (0, 0)),
    ]
    out_specs = (
        pl.BlockSpec((G, OUT), lambda i: (0, 0)),
        pl.BlockSpec((G, H), lambda i: (0, 0)),
    )
    out_shape = (
        jax.ShapeDtypeStruct((G, OUT), jnp.float32),
        jax.ShapeDtypeStruct((G, H), jnp.float32),
    )
    out, _ = pl.pallas_call(
        _last_body,
        grid=(NB,), in_specs=in_specs, out_specs=out_specs, out_shape=out_shape,
    )(acc, yp, dis, lab3, bat3, W, b3, p0, p1, M1, mb1, M2, mb2)
    return out


_deg_kernel = _make_deg_kernel()
_agg_e = _make_agg_kernel(edge_split=True)
_agg_f = _make_agg_kernel(edge_split=False)


def kernel(x_feat, clustering_labels, edge_index, batch,
           W0, b0, W1, b1, W2, b2, M1, mb1, M2, mb2):
    row = edge_index[0].astype(jnp.int32)
    col = edge_index[1].astype(jnp.int32)
    padi = (jnp.arange(EPAD - E, dtype=jnp.int32) % (NPAD - N)) + N
    rowp = jnp.concatenate([row, padi])
    colp = jnp.concatenate([col, padi])
    xfp = jnp.pad(x_feat, ((0, NPAD - N), (0, 0)))
    lab3 = jnp.pad(clustering_labels.astype(jnp.int32),
                   (0, NPAD - N)).reshape(NB, BN, 1)
    bat3 = jnp.pad(batch.astype(jnp.int32), (0, NPAD - N),
                   constant_values=-1).reshape(NB, 1, BN)

    degc = _deg_kernel(colp).reshape(2, NPAD, 128)[:, :, 0:1]
    dis, y0 = _prep_tc(degc, xfp)

    acc0 = _agg_e(rowp, colp, y0).reshape(2, NPAD, 128)
    y1, p0 = _layer_tc(acc0, y0, dis, lab3, bat3,
                       W0, b0.reshape(C, 1, H), split=False)
    acc1 = _agg_f(rowp, colp, y1.reshape(2 * NPAD, H // 2)).reshape(2, NPAD, 128)
    y2, p1 = _layer_tc(acc1, y1, dis, lab3, bat3,
                       W1, b1.reshape(C, 1, H), split=True)
    acc2 = _agg_f(rowp, colp, y2.reshape(2 * NPAD, H // 2)).reshape(2, NPAD, 128)
    out = _last_tc(acc2, y2, dis, lab3, bat3,
                   W2, b2.reshape(C, 1, H), p0, p1,
                   M1, mb1.reshape(1, H), M2, mb2.reshape(1, OUT))
    return out


# speedup vs baseline: 125.1662x; 1.0011x over previous
import functools

import jax
import jax.numpy as jnp
from jax import lax
from jax.experimental import pallas as pl
from jax.experimental.pallas import tpu as pltpu
from jax.experimental.pallas import tpu_sc as plsc

N = 10000
E = 320000
C = 8
IN = 128
H = 256
OUT = 128
G = 16

NPAD = 10240
EPAD = 327680
BN = 512
NB = NPAD // BN
CH = 128
NTILE = 16
RPT = NPAD // NTILE

_SC_MESH = dict(core_axis_name="c", subcore_axis_name="s")


def _zero_buf(buf, d):
    def zrow(r, _):
        for j in range(d // 16):
            buf[r, pl.ds(j * 16, 16)] = jnp.zeros((16,), jnp.float32)
        return 0

    lax.fori_loop(0, CH, zrow, 0)


def _zero_acc(buf, acc_sh, sid):
    base_r = sid * RPT
    for j in range(RPT // CH):
        pltpu.sync_copy(buf, acc_sh.at[pl.ds(base_r + j * CH, CH)])


def _make_deg_kernel():
    ept = EPAD // 2 // NTILE
    nchunk = ept // CH

    @functools.partial(
        pl.kernel,
        mesh=plsc.VectorSubcoreMesh(**_SC_MESH),
        out_type=jax.ShapeDtypeStruct((2 * NPAD, 128), jnp.float32),
        scratch_types=[
            pltpu.VMEM((EPAD // 2 // NTILE,), jnp.int32),
            pltpu.VMEM((CH, 128), jnp.float32),
            pltpu.VMEM_SHARED((NPAD, 128), jnp.float32),
        ],
    )
    def deg_kernel(col_hbm, out_hbm, idxc, vbuf, deg_sh):
        cid = lax.axis_index("c")
        sid = lax.axis_index("s")
        _zero_buf(vbuf, 128)
        _zero_acc(vbuf, deg_sh, sid)

        def orow(r, _):
            for j in range(128 // 16):
                vbuf[r, pl.ds(j * 16, 16)] = jnp.full((16,), 1.0, jnp.float32)
            return 0

        lax.fori_loop(0, CH, orow, 0)
        ebase = cid * (EPAD // 2) + sid * ept
        pltpu.sync_copy(col_hbm.at[pl.ds(ebase, ept)], idxc)
        plsc.subcore_barrier()

        def chunk(g, _):
            pltpu.sync_copy(vbuf, deg_sh.at[idxc.at[pl.ds(g * CH, CH)]],
                            add=True)
            return 0

        lax.fori_loop(0, nchunk, chunk, 0)
        plsc.subcore_barrier()
        base_r = sid * RPT
        pltpu.sync_copy(
            deg_sh.at[pl.ds(base_r, RPT)],
            out_hbm.at[pl.ds(cid * NPAD + base_r, RPT)],
        )

    return deg_kernel


def _make_agg_kernel(edge_split):
    ept = EPAD // NTILE // (2 if edge_split else 1)
    nchunk = ept // CH

    @functools.partial(
        pl.kernel,
        mesh=plsc.VectorSubcoreMesh(**_SC_MESH),
        out_type=jax.ShapeDtypeStruct((2 * NPAD, 128), jnp.float32),
        scratch_types=(
            [pltpu.VMEM((CH,), jnp.int32)] * 8
            + [pltpu.VMEM((CH, 128), jnp.float32)] * 2
            + [pltpu.VMEM_SHARED((NPAD, 128), jnp.float32)]
            + [pltpu.SemaphoreType.DMA] * 10
        ),
    )
    def agg_kernel(row_hbm, col_hbm, y_hbm, out_hbm,
                   ir0, ir1, ir2, ir3, ic0, ic1, ic2, ic3, buf0, buf1,
                   acc_sh, g0s, g1s, r0s, r1s, r2s, r3s, c0s, c1s, c2s, c3s):
        cid = lax.axis_index("c")
        sid = lax.axis_index("s")
        _zero_buf(buf0, 128)
        _zero_acc(buf0, acc_sh, sid)
        plsc.subcore_barrier()

        if edge_split:
            rbase = cid * (EPAD // 2) + sid * ept
            cbase = rbase
        else:
            rbase = cid * EPAD + sid * ept
            cbase = sid * ept

        idxr = (ir0, ir1, ir2, ir3)
        idxc = (ic0, ic1, ic2, ic3)
        buf = (buf0, buf1)
        gsem = (g0s, g1s)
        rsem = (r0s, r1s, r2s, r3s)
        csem = (c0s, c1s, c2s, c3s)

        def idx_load(g, s):
            return (pltpu.make_async_copy(
                        row_hbm.at[pl.ds(rbase + g * CH, CH)],
                        idxr[s], rsem[s]),
                    pltpu.make_async_copy(
                        col_hbm.at[pl.ds(cbase + g * CH, CH)],
                        idxc[s], csem[s]))

        def idx_start(g, s):
            a, b = idx_load(g, s)
            a.start()
            b.start()

        def idx_wait(g, s):
            a, b = idx_load(g, s)
            a.wait()
            b.wait()

        def gather(s, bs):
            return pltpu.make_async_copy(y_hbm.at[idxr[s]], buf[bs], gsem[bs])

        idx_start(0, 0)
        idx_wait(0, 0)
        gather(0, 0).start()
        idx_start(1, 1)

        def quad(i, _):
            q = 4 * i
            for k in range(4):
                g = q + k
                is_ = k
                bs = k % 2

                def stage_a(g=g, s=(k + 2) % 4):
                    idx_start(g + 2, s)

                def stage_b(g=g, s=(k + 1) % 4, nb=(k + 1) % 2):
                    idx_wait(g + 1, s)
                    gather(s, nb).start()

                if k < 2:
                    stage_a()
                else:
                    pl.when(g + 2 < nchunk)(stage_a)
                if k < 3:
                    stage_b()
                else:
                    pl.when(g + 1 < nchunk)(stage_b)
                gather(is_, bs).wait()
                pltpu.sync_copy(buf[bs], acc_sh.at[idxc[is_]], add=True)
            return 0

        lax.fori_loop(0, nchunk // 4, quad, 0)
        plsc.subcore_barrier()
        base_r = sid * RPT
        pltpu.sync_copy(
            acc_sh.at[pl.ds(base_r, RPT)],
            out_hbm.at[pl.ds(cid * NPAD + base_r, RPT)],
        )

    return agg_kernel


def _prep_body(degc_ref, xf_ref, dis_ref, y_ref):
    deg = degc_ref[0] + degc_ref[1] + 1.0
    dis = lax.rsqrt(deg)
    dis_ref[...] = dis
    y_ref[...] = dis * xf_ref[...]


def _prep_tc(degc, xfp):
    return pl.pallas_call(
        _prep_body,
        out_shape=(
            jax.ShapeDtypeStruct((NPAD, 1), jnp.float32),
            jax.ShapeDtypeStruct((NPAD, IN), jnp.float32),
        ),
    )(degc, xfp)


def _layer_body(acc_ref, yp_ref, dis_ref, lab_ref, bat_ref, W_ref, b_ref,
                yn_ref, pooled_ref, *, split, last):
    i = pl.program_id(0)
    dis = dis_ref[...]
    if split:
        agg = jnp.concatenate(
            [acc_ref[0] + yp_ref[0], acc_ref[1] + yp_ref[1]], axis=1) * dis
    else:
        agg = (acc_ref[0] + acc_ref[1] + yp_ref[...]) * dis
    lab = lab_ref[0]
    aggh = agg.astype(jnp.bfloat16)
    x2 = jnp.zeros((BN, H), jnp.float32)
    for c in range(C):
        v = jnp.dot(aggh, W_ref[c].astype(jnp.bfloat16),
                    preferred_element_type=jnp.float32) + b_ref[c]
        x2 = jnp.where(lab == c, v, x2)
    bat = bat_ref[0]
    oh = (lax.broadcasted_iota(jnp.int32, (G, BN), 0) == bat
          ).astype(jnp.float32)

    @pl.when(i == 0)
    def _():
        pooled_ref[...] = jnp.zeros((G, H), jnp.float32)

    pooled_ref[...] += jnp.dot(oh, x2, preferred_element_type=jnp.float32)
    if not last:
        yn = x2 * dis
        yn_ref[0] = yn[:, : H // 2]
        yn_ref[1] = yn[:, H // 2 :]


def _common_specs(W, b3, split):
    yp_spec = (pl.BlockSpec((2, BN, 128), lambda i: (0, i, 0)) if split
               else pl.BlockSpec((BN, 128), lambda i: (i, 0)))
    return [
        pl.BlockSpec((2, BN, 128), lambda i: (0, i, 0)),
        yp_spec,
        pl.BlockSpec((BN, 1), lambda i: (i, 0)),
        pl.BlockSpec((1, BN, 1), lambda i: (i, 0, 0)),
        pl.BlockSpec((1, 1, BN), lambda i: (i, 0, 0)),
        pl.BlockSpec(W.shape, lambda i: (0, 0, 0)),
        pl.BlockSpec(b3.shape, lambda i: (0, 0, 0)),
    ]


def _layer_tc(acc, yp, dis, lab3, bat3, W, b3, split):
    out_specs = (
        pl.BlockSpec((2, BN, H // 2), lambda i: (0, i, 0)),
        pl.BlockSpec((G, H), lambda i: (0, 0)),
    )
    out_shape = (
        jax.ShapeDtypeStruct((2, NPAD, H // 2), jnp.float32),
        jax.ShapeDtypeStruct((G, H), jnp.float32),
    )
    return pl.pallas_call(
        functools.partial(_layer_body, split=split, last=False),
        grid=(NB,), in_specs=_common_specs(W, b3, split),
        out_specs=out_specs, out_shape=out_shape,
    )(acc, yp, dis, lab3, bat3, W, b3)


def _last_body(acc_ref, yp_ref, dis_ref, lab_ref, bat_ref, W_ref, b_ref,
               p0_ref, p1_ref, M1_ref, mb1_ref, M2_ref, mb2_ref,
               out_ref, pooled_ref):
    _layer_body(acc_ref, yp_ref, dis_ref, lab_ref, bat_ref, W_ref, b_ref,
                None, pooled_ref, split=True, last=True)
    i = pl.program_id(0)

    @pl.when(i == NB - 1)
    def _():
        h = jnp.concatenate([p0_ref[...], p1_ref[...], pooled_ref[...]], axis=1)
        hh = jnp.maximum(
            jnp.dot(h, M1_ref[...], preferred_element_type=jnp.float32)
            + mb1_ref[...], 0.0)
        out_ref[...] = (jnp.dot(hh, M2_ref[...], preferred_element_type=jnp.float32)
                        + mb2_ref[...])


def _last_tc(acc, yp, dis, lab3, bat3, W, b3, p0, p1, M1, mb1, M2, mb2):
    in_specs = _common_specs(W, b3, True) + [
        pl.BlockSpec((G, H), lambda i: (0, 0)),
        pl.BlockSpec((G, H), lambda i: (0, 0)),
        pl.BlockSpec(M1.shape, lambda i: (0, 0)),
        pl.BlockSpec((1, H), lambda i: (0, 0)),
        pl.BlockSpec(M2.shape, lambda i: (0, 0)),
        pl.BlockSpec((1, OUT), lambda i: (0, 0)),
    ]
    out_specs = (
        pl.BlockSpec((G, OUT), lambda i: (0, 0)),
        pl.BlockSpec((G, H), lambda i: (0, 0)),
    )
    out_shape = (
        jax.ShapeDtypeStruct((G, OUT), jnp.float32),
        jax.ShapeDtypeStruct((G, H), jnp.float32),
    )
    out, _ = pl.pallas_call(
        _last_body,
        grid=(NB,), in_specs=in_specs, out_specs=out_specs, out_shape=out_shape,
    )(acc, yp, dis, lab3, bat3, W, b3, p0, p1, M1, mb1, M2, mb2)
    return out


_deg_kernel = _make_deg_kernel()
_agg_e = _make_agg_kernel(edge_split=True)
_agg_f = _make_agg_kernel(edge_split=False)


def kernel(x_feat, clustering_labels, edge_index, batch,
           W0, b0, W1, b1, W2, b2, M1, mb1, M2, mb2):
    row = edge_index[0].astype(jnp.int32)
    col = edge_index[1].astype(jnp.int32)
    padi = (jnp.arange(EPAD - E, dtype=jnp.int32) % (NPAD - N)) + N
    rowp1 = jnp.concatenate([row, padi])
    rowp = jnp.concatenate([rowp1, rowp1 + NPAD])
    colp = jnp.concatenate([col, padi])
    xfp = jnp.pad(x_feat, ((0, NPAD - N), (0, 0)))
    lab3 = jnp.pad(clustering_labels.astype(jnp.int32),
                   (0, NPAD - N)).reshape(NB, BN, 1)
    bat3 = jnp.pad(batch.astype(jnp.int32), (0, NPAD - N),
                   constant_values=-1).reshape(NB, 1, BN)

    degc = _deg_kernel(colp).reshape(2, NPAD, 128)[:, :, 0:1]
    dis, y0 = _prep_tc(degc, xfp)

    acc0 = _agg_e(rowp, colp, y0).reshape(2, NPAD, 128)
    y1, p0 = _layer_tc(acc0, y0, dis, lab3, bat3,
                       W0, b0.reshape(C, 1, H), split=False)
    acc1 = _agg_f(rowp, colp, y1.reshape(2 * NPAD, H // 2)).reshape(2, NPAD, 128)
    y2, p1 = _layer_tc(acc1, y1, dis, lab3, bat3,
                       W1, b1.reshape(C, 1, H), split=True)
    acc2 = _agg_f(rowp, colp, y2.reshape(2 * NPAD, H // 2)).reshape(2, NPAD, 128)
    out = _last_tc(acc2, y2, dis, lab3, bat3,
                   W2, b2.reshape(C, 1, H), p0, p1,
                   M1, mb1.reshape(1, H), M2, mb2.reshape(1, OUT))
    return out
